# baseline (speedup 1.0000x reference)
import functools

import jax
import jax.numpy as jnp
from jax import lax
from jax.experimental import pallas as pl
from jax.experimental.pallas import tpu as pltpu
from jax.experimental.pallas import tpu_sc as plsc

NB = 256
GP = 32


def _dotT(x, w):
    return jax.lax.dot_general(x, w, (((1,), (1,)), ((), ())),
                               preferred_element_type=jnp.float32)


def _k1_body(tx_ref, sxT_ref, idx_ref, w_ref):
    S = sxT_ref.shape[2]
    NBb = tx_ref.shape[1]
    t = tx_ref[0]
    s = sxT_ref[0]
    cross = jnp.dot(t, s, preferred_element_type=jnp.float32)
    t2 = jnp.sum(t * t, axis=1, keepdims=True)
    s2 = jnp.sum(s * s, axis=0, keepdims=True)
    ds = jnp.clip(t2 + s2 - 2.0 * cross, 0.0, None)
    iota = jax.lax.broadcasted_iota(jnp.int32, (NBb, S), 1)
    dm = ds
    idxs, ws = [], []
    for k in range(3):
        m = jnp.min(dm, axis=1, keepdims=True)
        ik = jnp.min(jnp.where(dm <= m, iota, S), axis=1, keepdims=True)
        selk = iota == ik
        ck = jax.lax.dot_general(
            selk.astype(jnp.float32), s, (((1,), (1,)), ((), ())),
            precision=jax.lax.Precision.HIGHEST,
            preferred_element_type=jnp.float32)
        diff = t - ck
        wd = jnp.sum(diff * diff, axis=1, keepdims=True)
        ws.append(1.0 / (wd + 1e-8))
        idxs.append(ik)
        if k < 2:
            dm = jnp.where(selk, jnp.inf, dm)
    rs = ws[0] + ws[1] + ws[2]
    b = pl.program_id(0)
    idx_ref[0] = jnp.concatenate(idxs, axis=1) + b * S
    w_ref[0] = jnp.concatenate([w / rs for w in ws], axis=1)


def _make_sc_gather(PTS, C2, NW):
    PPW = PTS // NW
    NG = PPW // GP
    GPC = GP * 3
    mesh = plsc.VectorSubcoreMesh(core_axis_name="c", subcore_axis_name="s")
    NC = 2

    @functools.partial(
        pl.kernel, mesh=mesh,
        compiler_params=pltpu.CompilerParams(needs_layout_passes=False),
        out_type=jax.ShapeDtypeStruct((PTS * 3, C2), jnp.float32),
        scratch_types=[
            pltpu.VMEM((NG, GPC), jnp.int32),
            pltpu.VMEM((GPC, C2), jnp.float32),
            pltpu.VMEM((GPC, C2), jnp.float32),
            pltpu.SemaphoreType.DMA,
            pltpu.SemaphoreType.DMA,
        ],
    )
    def sc_gather(idx_hbm, f_hbm, out_hbm, idx_v, buf0, buf1, sem0, sem1):
        wid = lax.axis_index("s") * NC + lax.axis_index("c")
        pltpu.sync_copy(idx_hbm.at[pl.ds(wid * NG, NG)], idx_v)
        pltpu.async_copy(f_hbm.at[idx_v.at[0]], buf0, sem0)
        pltpu.async_copy(f_hbm.at[idx_v.at[1]], buf1, sem1)
        obase = wid * NG

        def pair(t, carry):
            g = t * 2
            for b in range(2):
                buf = buf0 if b == 0 else buf1
                sem = sem0 if b == 0 else sem1
                gg = g + b
                pltpu.make_async_copy(f_hbm.at[idx_v.at[0]], buf, sem).wait()
                pltpu.sync_copy(
                    buf, out_hbm.at[pl.ds((obase + gg) * GPC, GPC)])

                @pl.when(gg + 2 < NG)
                def _():
                    pltpu.async_copy(f_hbm.at[idx_v.at[gg + 2]], buf, sem)
            return carry

        lax.fori_loop(0, NG // 2, pair, 0)

    return sc_gather


def _k2_body(rows_ref, w_ref, skip_ref, W1_ref, y1_ref, stats_ref):
    g = rows_ref[0]
    w = w_ref[0]
    interp = (w[:, 0:1] * g[:, :256] + w[:, 1:2] * g[:, 256:512]
              + w[:, 2:3] * g[:, 512:])
    y1 = (_dotT(interp, W1_ref[:, :256])
          + _dotT(skip_ref[0], W1_ref[:, 256:]))
    y1_ref[0] = y1
    st = jnp.concatenate([jnp.sum(y1, axis=0)[None, :],
                          jnp.sum(y1 * y1, axis=0)[None, :]], axis=0)
    first = (pl.program_id(0) == 0) & (pl.program_id(1) == 0)

    @pl.when(first)
    def _():
        stats_ref[...] = st

    @pl.when(jnp.logical_not(first))
    def _():
        stats_ref[...] += st


def _k3_body(y1_ref, a1_ref, c1_ref, W2_ref, y2_ref, stats_ref):
    z = jnp.maximum(y1_ref[0] * a1_ref[...] + c1_ref[...], 0.0)
    y2 = _dotT(z, W2_ref[...])
    y2_ref[0] = y2
    st = jnp.concatenate([jnp.sum(y2, axis=0)[None, :],
                          jnp.sum(y2 * y2, axis=0)[None, :]], axis=0)
    first = (pl.program_id(0) == 0) & (pl.program_id(1) == 0)

    @pl.when(first)
    def _():
        stats_ref[...] = st

    @pl.when(jnp.logical_not(first))
    def _():
        stats_ref[...] += st


def _k4_body(y2_ref, a2_ref, c2_ref, out_ref):
    out_ref[0] = jnp.maximum(y2_ref[0] * a2_ref[...] + c2_ref[...], 0.0)


def kernel(target_xyz, source_xyz, source_features, target_skip_features,
           W1, g1, b1, W2, g2, b2):
    B, N, _ = target_xyz.shape
    S = source_xyz.shape[1]
    C2 = source_features.shape[2]
    C1 = target_skip_features.shape[2]
    nblk = N // NB
    PTS = B * N
    sxT = jnp.transpose(source_xyz, (0, 2, 1))

    gidx, wn = pl.pallas_call(
        _k1_body,
        grid=(B, nblk),
        in_specs=[
            pl.BlockSpec((1, NB, 3), lambda b, n: (b, n, 0)),
            pl.BlockSpec((1, 3, S), lambda b, n: (b, 0, 0)),
        ],
        out_specs=[
            pl.BlockSpec((1, NB, 3), lambda b, n: (b, n, 0)),
            pl.BlockSpec((1, NB, 3), lambda b, n: (b, n, 0)),
        ],
        out_shape=[
            jax.ShapeDtypeStruct((B, N, 3), jnp.int32),
            jax.ShapeDtypeStruct((B, N, 3), jnp.float32),
        ],
    )(target_xyz, sxT)

    sc_gather = _make_sc_gather(PTS, C2, 32)
    rows = sc_gather(gidx.reshape(-1, GP * 3),
                     source_features.reshape(B * S, C2))
    rows = rows.reshape(B, N, 3 * C2)

    y1, st1 = pl.pallas_call(
        _k2_body,
        grid=(B, nblk),
        in_specs=[
            pl.BlockSpec((1, NB, 3 * C2), lambda b, n: (b, n, 0)),
            pl.BlockSpec((1, NB, 3), lambda b, n: (b, n, 0)),
            pl.BlockSpec((1, NB, C1), lambda b, n: (b, n, 0)),
            pl.BlockSpec((256, 384), lambda b, n: (0, 0)),
        ],
        out_specs=[
            pl.BlockSpec((1, NB, 256), lambda b, n: (b, n, 0)),
            pl.BlockSpec((2, 256), lambda b, n: (0, 0)),
        ],
        out_shape=[
            jax.ShapeDtypeStruct((B, N, 256), jnp.float32),
            jax.ShapeDtypeStruct((2, 256), jnp.float32),
        ],
    )(rows, wn, target_skip_features, W1)

    cnt = float(B * N)
    mean1 = st1[0] / cnt
    var1 = st1[1] / cnt - mean1 * mean1
    a1 = g1 * jax.lax.rsqrt(var1 + 1e-5)
    c1 = b1 - mean1 * a1

    y2, st2 = pl.pallas_call(
        _k3_body,
        grid=(B, nblk),
        in_specs=[
            pl.BlockSpec((1, NB, 256), lambda b, n: (b, n, 0)),
            pl.BlockSpec((1, 256), lambda b, n: (0, 0)),
            pl.BlockSpec((1, 256), lambda b, n: (0, 0)),
            pl.BlockSpec((128, 256), lambda b, n: (0, 0)),
        ],
        out_specs=[
            pl.BlockSpec((1, NB, 128), lambda b, n: (b, n, 0)),
            pl.BlockSpec((2, 128), lambda b, n: (0, 0)),
        ],
        out_shape=[
            jax.ShapeDtypeStruct((B, N, 128), jnp.float32),
            jax.ShapeDtypeStruct((2, 128), jnp.float32),
        ],
    )(y1, a1[None, :], c1[None, :], W2)

    mean2 = st2[0] / cnt
    var2 = st2[1] / cnt - mean2 * mean2
    a2 = g2 * jax.lax.rsqrt(var2 + 1e-5)
    c2 = b2 - mean2 * a2

    out = pl.pallas_call(
        _k4_body,
        grid=(B, nblk),
        in_specs=[
            pl.BlockSpec((1, NB, 128), lambda b, n: (b, n, 0)),
            pl.BlockSpec((1, 128), lambda b, n: (0, 0)),
            pl.BlockSpec((1, 128), lambda b, n: (0, 0)),
        ],
        out_specs=pl.BlockSpec((1, NB, 128), lambda b, n: (b, n, 0)),
        out_shape=jax.ShapeDtypeStruct((B, N, 128), jnp.float32),
    )(y2, a2[None, :], c2[None, :])
    return out


# speedup vs baseline: 1.3450x; 1.3450x over previous
import functools

import jax
import jax.numpy as jnp
from jax import lax
from jax.experimental import pallas as pl
from jax.experimental.pallas import tpu as pltpu
from jax.experimental.pallas import tpu_sc as plsc

NB = 256
GP = 32


def _dotT(x, w):
    return jax.lax.dot_general(x, w, (((1,), (1,)), ((), ())),
                               preferred_element_type=jnp.float32)


def _k1_body(tx_ref, sxT_ref, idx_ref, w_ref):
    S = sxT_ref.shape[2]
    NBb = tx_ref.shape[1]
    t = tx_ref[0]
    s = sxT_ref[0]
    cross = jnp.dot(t, s, preferred_element_type=jnp.float32)
    t2 = jnp.sum(t * t, axis=1, keepdims=True)
    s2 = jnp.sum(s * s, axis=0, keepdims=True)
    ds = jnp.clip(t2 + s2 - 2.0 * cross, 0.0, None)
    dd = None
    for c in range(3):
        diff = t[:, c:c + 1] - s[c:c + 1, :]
        dd = diff * diff if dd is None else dd + diff * diff
    iota = jax.lax.broadcasted_iota(jnp.int32, (NBb, S), 1)
    dm = ds
    idxs, ws = [], []
    for k in range(3):
        m = jnp.min(dm, axis=1, keepdims=True)
        ik = jnp.min(jnp.where(dm <= m, iota, S), axis=1, keepdims=True)
        selk = iota == ik
        wd = jnp.sum(jnp.where(selk, dd, 0.0), axis=1, keepdims=True)
        ws.append(1.0 / (wd + 1e-8))
        idxs.append(ik)
        if k < 2:
            dm = jnp.where(selk, jnp.inf, dm)
    rs = ws[0] + ws[1] + ws[2]
    b = pl.program_id(0)
    idx_ref[0] = jnp.concatenate(idxs, axis=1) + b * S
    w_ref[0] = jnp.concatenate([w / rs for w in ws], axis=1)


def _make_sc_gather(PTS, C2, NW):
    PPW = PTS // NW
    NG = PPW // GP
    GPC = GP * 3
    mesh = plsc.VectorSubcoreMesh(core_axis_name="c", subcore_axis_name="s")
    NC = 2

    @functools.partial(
        pl.kernel, mesh=mesh,
        compiler_params=pltpu.CompilerParams(needs_layout_passes=False),
        out_type=jax.ShapeDtypeStruct((PTS * 3, C2), jnp.float32),
        scratch_types=[
            pltpu.VMEM((NG, GPC), jnp.int32),
            pltpu.VMEM((GPC, C2), jnp.float32),
            pltpu.VMEM((GPC, C2), jnp.float32),
            pltpu.SemaphoreType.DMA,
            pltpu.SemaphoreType.DMA,
        ],
    )
    def sc_gather(idx_hbm, f_hbm, out_hbm, idx_v, buf0, buf1, sem0, sem1):
        wid = lax.axis_index("s") * NC + lax.axis_index("c")
        pltpu.sync_copy(idx_hbm.at[pl.ds(wid * NG, NG)], idx_v)
        pltpu.async_copy(f_hbm.at[idx_v.at[0]], buf0, sem0)
        pltpu.async_copy(f_hbm.at[idx_v.at[1]], buf1, sem1)
        obase = wid * NG

        def pair(t, carry):
            g = t * 2
            for b in range(2):
                buf = buf0 if b == 0 else buf1
                sem = sem0 if b == 0 else sem1
                gg = g + b
                pltpu.make_async_copy(f_hbm.at[idx_v.at[0]], buf, sem).wait()
                pltpu.sync_copy(
                    buf, out_hbm.at[pl.ds((obase + gg) * GPC, GPC)])

                @pl.when(gg + 2 < NG)
                def _():
                    pltpu.async_copy(f_hbm.at[idx_v.at[gg + 2]], buf, sem)
            return carry

        lax.fori_loop(0, NG // 2, pair, 0)

    return sc_gather


def _k2_body(rows_ref, w_ref, skip_ref, W1_ref, y1_ref, stats_ref):
    g = rows_ref[0]
    w = w_ref[0]
    interp = (w[:, 0:1] * g[:, :256] + w[:, 1:2] * g[:, 256:512]
              + w[:, 2:3] * g[:, 512:])
    y1 = (_dotT(interp, W1_ref[:, :256])
          + _dotT(skip_ref[0], W1_ref[:, 256:]))
    y1_ref[0] = y1
    st = jnp.concatenate([jnp.sum(y1, axis=0)[None, :],
                          jnp.sum(y1 * y1, axis=0)[None, :]], axis=0)
    first = (pl.program_id(0) == 0) & (pl.program_id(1) == 0)

    @pl.when(first)
    def _():
        stats_ref[...] = st

    @pl.when(jnp.logical_not(first))
    def _():
        stats_ref[...] += st


def _k3_body(y1_ref, a1_ref, c1_ref, W2_ref, y2_ref, stats_ref):
    z = jnp.maximum(y1_ref[0] * a1_ref[...] + c1_ref[...], 0.0)
    y2 = _dotT(z, W2_ref[...])
    y2_ref[0] = y2
    st = jnp.concatenate([jnp.sum(y2, axis=0)[None, :],
                          jnp.sum(y2 * y2, axis=0)[None, :]], axis=0)
    first = (pl.program_id(0) == 0) & (pl.program_id(1) == 0)

    @pl.when(first)
    def _():
        stats_ref[...] = st

    @pl.when(jnp.logical_not(first))
    def _():
        stats_ref[...] += st


def _k4_body(y2_ref, a2_ref, c2_ref, out_ref):
    out_ref[0] = jnp.maximum(y2_ref[0] * a2_ref[...] + c2_ref[...], 0.0)


def kernel(target_xyz, source_xyz, source_features, target_skip_features,
           W1, g1, b1, W2, g2, b2):
    B, N, _ = target_xyz.shape
    S = source_xyz.shape[1]
    C2 = source_features.shape[2]
    C1 = target_skip_features.shape[2]
    nblk = N // NB
    PTS = B * N
    sxT = jnp.transpose(source_xyz, (0, 2, 1))

    gidx, wn = pl.pallas_call(
        _k1_body,
        grid=(B, nblk),
        in_specs=[
            pl.BlockSpec((1, NB, 3), lambda b, n: (b, n, 0)),
            pl.BlockSpec((1, 3, S), lambda b, n: (b, 0, 0)),
        ],
        out_specs=[
            pl.BlockSpec((1, NB, 3), lambda b, n: (b, n, 0)),
            pl.BlockSpec((1, NB, 3), lambda b, n: (b, n, 0)),
        ],
        out_shape=[
            jax.ShapeDtypeStruct((B, N, 3), jnp.int32),
            jax.ShapeDtypeStruct((B, N, 3), jnp.float32),
        ],
    )(target_xyz, sxT)

    sc_gather = _make_sc_gather(PTS, C2, 32)
    rows = sc_gather(gidx.reshape(-1, GP * 3),
                     source_features.reshape(B * S, C2))
    rows = rows.reshape(B, N, 3 * C2)

    y1, st1 = pl.pallas_call(
        _k2_body,
        grid=(B, nblk),
        in_specs=[
            pl.BlockSpec((1, NB, 3 * C2), lambda b, n: (b, n, 0)),
            pl.BlockSpec((1, NB, 3), lambda b, n: (b, n, 0)),
            pl.BlockSpec((1, NB, C1), lambda b, n: (b, n, 0)),
            pl.BlockSpec((256, 384), lambda b, n: (0, 0)),
        ],
        out_specs=[
            pl.BlockSpec((1, NB, 256), lambda b, n: (b, n, 0)),
            pl.BlockSpec((2, 256), lambda b, n: (0, 0)),
        ],
        out_shape=[
            jax.ShapeDtypeStruct((B, N, 256), jnp.float32),
            jax.ShapeDtypeStruct((2, 256), jnp.float32),
        ],
    )(rows, wn, target_skip_features, W1)

    cnt = float(B * N)
    mean1 = st1[0] / cnt
    var1 = st1[1] / cnt - mean1 * mean1
    a1 = g1 * jax.lax.rsqrt(var1 + 1e-5)
    c1 = b1 - mean1 * a1

    y2, st2 = pl.pallas_call(
        _k3_body,
        grid=(B, nblk),
        in_specs=[
            pl.BlockSpec((1, NB, 256), lambda b, n: (b, n, 0)),
            pl.BlockSpec((1, 256), lambda b, n: (0, 0)),
            pl.BlockSpec((1, 256), lambda b, n: (0, 0)),
            pl.BlockSpec((128, 256), lambda b, n: (0, 0)),
        ],
        out_specs=[
            pl.BlockSpec((1, NB, 128), lambda b, n: (b, n, 0)),
            pl.BlockSpec((2, 128), lambda b, n: (0, 0)),
        ],
        out_shape=[
            jax.ShapeDtypeStruct((B, N, 128), jnp.float32),
            jax.ShapeDtypeStruct((2, 128), jnp.float32),
        ],
    )(y1, a1[None, :], c1[None, :], W2)

    mean2 = st2[0] / cnt
    var2 = st2[1] / cnt - mean2 * mean2
    a2 = g2 * jax.lax.rsqrt(var2 + 1e-5)
    c2 = b2 - mean2 * a2

    out = pl.pallas_call(
        _k4_body,
        grid=(B, nblk),
        in_specs=[
            pl.BlockSpec((1, NB, 128), lambda b, n: (b, n, 0)),
            pl.BlockSpec((1, 128), lambda b, n: (0, 0)),
            pl.BlockSpec((1, 128), lambda b, n: (0, 0)),
        ],
        out_specs=pl.BlockSpec((1, NB, 128), lambda b, n: (b, n, 0)),
        out_shape=jax.ShapeDtypeStruct((B, N, 128), jnp.float32),
    )(y2, a2[None, :], c2[None, :])
    return out


# speedup vs baseline: 1.6212x; 1.2054x over previous
import functools

import jax
import jax.numpy as jnp
from jax import lax
from jax.experimental import pallas as pl
from jax.experimental.pallas import tpu as pltpu
from jax.experimental.pallas import tpu_sc as plsc

NB = 512
GP = 32


def _dotT(x, w):
    return jax.lax.dot_general(x, w, (((1,), (1,)), ((), ())),
                               preferred_element_type=jnp.float32)


def _k1_body(tx_ref, sxT_ref, idx_ref, w_ref):
    S = sxT_ref.shape[2]
    NBb = tx_ref.shape[1]
    t = tx_ref[0]
    s = sxT_ref[0]
    cross = jnp.dot(t, s, preferred_element_type=jnp.float32)
    t2 = jnp.sum(t * t, axis=1, keepdims=True)
    s2 = jnp.sum(s * s, axis=0, keepdims=True)
    ds = jnp.clip(t2 + s2 - 2.0 * cross, 0.0, None)
    dd = None
    for c in range(3):
        diff = t[:, c:c + 1] - s[c:c + 1, :]
        dd = diff * diff if dd is None else dd + diff * diff
    iota = jax.lax.broadcasted_iota(jnp.int32, (NBb, S), 1)
    dm = ds
    idxs, ws = [], []
    for k in range(3):
        m = jnp.min(dm, axis=1, keepdims=True)
        ik = jnp.min(jnp.where(dm <= m, iota, S), axis=1, keepdims=True)
        selk = iota == ik
        wd = jnp.sum(jnp.where(selk, dd, 0.0), axis=1, keepdims=True)
        ws.append(1.0 / (wd + 1e-8))
        idxs.append(ik)
        if k < 2:
            dm = jnp.where(selk, jnp.inf, dm)
    rs = ws[0] + ws[1] + ws[2]
    b = pl.program_id(0)
    idx_ref[0] = jnp.concatenate(idxs, axis=1) + b * S
    w_ref[0] = jnp.concatenate([w / rs for w in ws], axis=1)


def _make_sc_gather(PTS, C2, NW):
    PPW = PTS // NW
    NG = PPW // GP
    GPC = GP * 3
    mesh = plsc.VectorSubcoreMesh(core_axis_name="c", subcore_axis_name="s")
    NC = 2

    @functools.partial(
        pl.kernel, mesh=mesh,
        compiler_params=pltpu.CompilerParams(needs_layout_passes=False),
        out_type=jax.ShapeDtypeStruct((PTS * 3, C2), jnp.float32),
        scratch_types=[
            pltpu.VMEM((NG, GPC), jnp.int32),
            pltpu.VMEM((GPC, C2), jnp.float32),
            pltpu.VMEM((GPC, C2), jnp.float32),
            pltpu.SemaphoreType.DMA,
            pltpu.SemaphoreType.DMA,
        ],
    )
    def sc_gather(idx_hbm, f_hbm, out_hbm, idx_v, buf0, buf1, sem0, sem1):
        wid = lax.axis_index("s") * NC + lax.axis_index("c")
        pltpu.sync_copy(idx_hbm.at[pl.ds(wid * NG, NG)], idx_v)
        pltpu.async_copy(f_hbm.at[idx_v.at[0]], buf0, sem0)
        pltpu.async_copy(f_hbm.at[idx_v.at[1]], buf1, sem1)
        obase = wid * NG

        def pair(t, carry):
            g = t * 2
            for b in range(2):
                buf = buf0 if b == 0 else buf1
                sem = sem0 if b == 0 else sem1
                gg = g + b
                pltpu.make_async_copy(f_hbm.at[idx_v.at[0]], buf, sem).wait()
                pltpu.sync_copy(
                    buf, out_hbm.at[pl.ds((obase + gg) * GPC, GPC)])

                @pl.when(gg + 2 < NG)
                def _():
                    pltpu.async_copy(f_hbm.at[idx_v.at[gg + 2]], buf, sem)
            return carry

        lax.fori_loop(0, NG // 2, pair, 0)

    return sc_gather


def _k2_body(rows_ref, w_ref, skip_ref, W1_ref, y1_ref, stats_ref):
    g = rows_ref[0]
    w = w_ref[0]
    interp = (w[:, 0:1] * g[:, :256] + w[:, 1:2] * g[:, 256:512]
              + w[:, 2:3] * g[:, 512:])
    y1 = (_dotT(interp, W1_ref[:, :256])
          + _dotT(skip_ref[0], W1_ref[:, 256:]))
    y1_ref[0] = y1
    st = jnp.concatenate([jnp.sum(y1, axis=0)[None, :],
                          jnp.sum(y1 * y1, axis=0)[None, :]], axis=0)
    first = (pl.program_id(0) == 0) & (pl.program_id(1) == 0)

    @pl.when(first)
    def _():
        stats_ref[...] = st

    @pl.when(jnp.logical_not(first))
    def _():
        stats_ref[...] += st


def _k3_body(y1_ref, a1_ref, c1_ref, W2_ref, y2_ref, stats_ref):
    z = jnp.maximum(y1_ref[0] * a1_ref[...] + c1_ref[...], 0.0)
    y2 = _dotT(z, W2_ref[...])
    y2_ref[0] = y2
    st = jnp.concatenate([jnp.sum(y2, axis=0)[None, :],
                          jnp.sum(y2 * y2, axis=0)[None, :]], axis=0)
    first = (pl.program_id(0) == 0) & (pl.program_id(1) == 0)

    @pl.when(first)
    def _():
        stats_ref[...] = st

    @pl.when(jnp.logical_not(first))
    def _():
        stats_ref[...] += st


def _k4_body(y2_ref, a2_ref, c2_ref, out_ref):
    out_ref[0] = jnp.maximum(y2_ref[0] * a2_ref[...] + c2_ref[...], 0.0)


def kernel(target_xyz, source_xyz, source_features, target_skip_features,
           W1, g1, b1, W2, g2, b2):
    B, N, _ = target_xyz.shape
    S = source_xyz.shape[1]
    C2 = source_features.shape[2]
    C1 = target_skip_features.shape[2]
    nblk = N // NB
    PTS = B * N
    sxT = jnp.transpose(source_xyz, (0, 2, 1))

    gidx, wn = pl.pallas_call(
        _k1_body,
        grid=(B, nblk),
        in_specs=[
            pl.BlockSpec((1, NB, 3), lambda b, n: (b, n, 0)),
            pl.BlockSpec((1, 3, S), lambda b, n: (b, 0, 0)),
        ],
        out_specs=[
            pl.BlockSpec((1, NB, 3), lambda b, n: (b, n, 0)),
            pl.BlockSpec((1, NB, 3), lambda b, n: (b, n, 0)),
        ],
        out_shape=[
            jax.ShapeDtypeStruct((B, N, 3), jnp.int32),
            jax.ShapeDtypeStruct((B, N, 3), jnp.float32),
        ],
    )(target_xyz, sxT)

    sc_gather = _make_sc_gather(PTS, C2, 32)
    rows = sc_gather(gidx.reshape(-1, GP * 3),
                     source_features.reshape(B * S, C2))
    rows = rows.reshape(B, N, 3 * C2)

    y1, st1 = pl.pallas_call(
        _k2_body,
        grid=(B, nblk),
        in_specs=[
            pl.BlockSpec((1, NB, 3 * C2), lambda b, n: (b, n, 0)),
            pl.BlockSpec((1, NB, 3), lambda b, n: (b, n, 0)),
            pl.BlockSpec((1, NB, C1), lambda b, n: (b, n, 0)),
            pl.BlockSpec((256, 384), lambda b, n: (0, 0)),
        ],
        out_specs=[
            pl.BlockSpec((1, NB, 256), lambda b, n: (b, n, 0)),
            pl.BlockSpec((2, 256), lambda b, n: (0, 0)),
        ],
        out_shape=[
            jax.ShapeDtypeStruct((B, N, 256), jnp.float32),
            jax.ShapeDtypeStruct((2, 256), jnp.float32),
        ],
    )(rows, wn, target_skip_features, W1)

    cnt = float(B * N)
    mean1 = st1[0] / cnt
    var1 = st1[1] / cnt - mean1 * mean1
    a1 = g1 * jax.lax.rsqrt(var1 + 1e-5)
    c1 = b1 - mean1 * a1

    y2, st2 = pl.pallas_call(
        _k3_body,
        grid=(B, nblk),
        in_specs=[
            pl.BlockSpec((1, NB, 256), lambda b, n: (b, n, 0)),
            pl.BlockSpec((1, 256), lambda b, n: (0, 0)),
            pl.BlockSpec((1, 256), lambda b, n: (0, 0)),
            pl.BlockSpec((128, 256), lambda b, n: (0, 0)),
        ],
        out_specs=[
            pl.BlockSpec((1, NB, 128), lambda b, n: (b, n, 0)),
            pl.BlockSpec((2, 128), lambda b, n: (0, 0)),
        ],
        out_shape=[
            jax.ShapeDtypeStruct((B, N, 128), jnp.float32),
            jax.ShapeDtypeStruct((2, 128), jnp.float32),
        ],
    )(y1, a1[None, :], c1[None, :], W2)

    mean2 = st2[0] / cnt
    var2 = st2[1] / cnt - mean2 * mean2
    a2 = g2 * jax.lax.rsqrt(var2 + 1e-5)
    c2 = b2 - mean2 * a2

    out = pl.pallas_call(
        _k4_body,
        grid=(B, nblk),
        in_specs=[
            pl.BlockSpec((1, NB, 128), lambda b, n: (b, n, 0)),
            pl.BlockSpec((1, 128), lambda b, n: (0, 0)),
            pl.BlockSpec((1, 128), lambda b, n: (0, 0)),
        ],
        out_specs=pl.BlockSpec((1, NB, 128), lambda b, n: (b, n, 0)),
        out_shape=jax.ShapeDtypeStruct((B, N, 128), jnp.float32),
    )(y2, a2[None, :], c2[None, :])
    return out


# speedup vs baseline: 1.7770x; 1.0961x over previous
import functools

import jax
import jax.numpy as jnp
from jax import lax
from jax.experimental import pallas as pl
from jax.experimental.pallas import tpu as pltpu
from jax.experimental.pallas import tpu_sc as plsc

NB = 1024
GP = 32


def _dotT(x, w):
    return jax.lax.dot_general(x, w, (((1,), (1,)), ((), ())),
                               preferred_element_type=jnp.float32)


def _k1_body(tx_ref, sxT_ref, idx_ref, w_ref):
    S = sxT_ref.shape[2]
    NBb = tx_ref.shape[1]
    t = tx_ref[0]
    s = sxT_ref[0]
    cross = jnp.dot(t, s, preferred_element_type=jnp.float32)
    t2 = jnp.sum(t * t, axis=1, keepdims=True)
    s2 = jnp.sum(s * s, axis=0, keepdims=True)
    ds = jnp.clip(t2 + s2 - 2.0 * cross, 0.0, None)
    dd = None
    for c in range(3):
        diff = t[:, c:c + 1] - s[c:c + 1, :]
        dd = diff * diff if dd is None else dd + diff * diff
    iota = jax.lax.broadcasted_iota(jnp.int32, (NBb, S), 1)
    dm = ds
    idxs, ws = [], []
    for k in range(3):
        m = jnp.min(dm, axis=1, keepdims=True)
        ik = jnp.min(jnp.where(dm <= m, iota, S), axis=1, keepdims=True)
        selk = iota == ik
        wd = jnp.sum(jnp.where(selk, dd, 0.0), axis=1, keepdims=True)
        ws.append(1.0 / (wd + 1e-8))
        idxs.append(ik)
        if k < 2:
            dm = jnp.where(selk, jnp.inf, dm)
    rs = ws[0] + ws[1] + ws[2]
    b = pl.program_id(0)
    idx_ref[0] = jnp.concatenate(idxs, axis=1) + b * S
    w_ref[0] = jnp.concatenate([w / rs for w in ws], axis=1)


def _make_sc_gather(PTS, C2, NW):
    PPW = PTS // NW
    NG = PPW // GP
    GPC = GP * 3
    mesh = plsc.VectorSubcoreMesh(core_axis_name="c", subcore_axis_name="s")
    NC = 2

    @functools.partial(
        pl.kernel, mesh=mesh,
        compiler_params=pltpu.CompilerParams(needs_layout_passes=False),
        out_type=jax.ShapeDtypeStruct((PTS * 3, C2), jnp.float32),
        scratch_types=[
            pltpu.VMEM((NG, GPC), jnp.int32),
            pltpu.VMEM((GPC, C2), jnp.float32),
            pltpu.VMEM((GPC, C2), jnp.float32),
            pltpu.SemaphoreType.DMA,
            pltpu.SemaphoreType.DMA,
        ],
    )
    def sc_gather(idx_hbm, f_hbm, out_hbm, idx_v, buf0, buf1, sem0, sem1):
        wid = lax.axis_index("s") * NC + lax.axis_index("c")
        pltpu.sync_copy(idx_hbm.at[pl.ds(wid * NG, NG)], idx_v)
        pltpu.async_copy(f_hbm.at[idx_v.at[0]], buf0, sem0)
        pltpu.async_copy(f_hbm.at[idx_v.at[1]], buf1, sem1)
        obase = wid * NG

        def pair(t, carry):
            g = t * 2
            for b in range(2):
                buf = buf0 if b == 0 else buf1
                sem = sem0 if b == 0 else sem1
                gg = g + b
                pltpu.make_async_copy(f_hbm.at[idx_v.at[0]], buf, sem).wait()
                pltpu.sync_copy(
                    buf, out_hbm.at[pl.ds((obase + gg) * GPC, GPC)])

                @pl.when(gg + 2 < NG)
                def _():
                    pltpu.async_copy(f_hbm.at[idx_v.at[gg + 2]], buf, sem)
            return carry

        lax.fori_loop(0, NG // 2, pair, 0)

    return sc_gather


def _k2_body(rows_ref, w_ref, skip_ref, W1_ref, y1_ref, stats_ref):
    g = rows_ref[0]
    w = w_ref[0]
    interp = (w[:, 0:1] * g[:, :256] + w[:, 1:2] * g[:, 256:512]
              + w[:, 2:3] * g[:, 512:])
    y1 = (_dotT(interp, W1_ref[:, :256])
          + _dotT(skip_ref[0], W1_ref[:, 256:]))
    y1_ref[0] = y1
    st = jnp.concatenate([jnp.sum(y1, axis=0)[None, :],
                          jnp.sum(y1 * y1, axis=0)[None, :]], axis=0)
    first = (pl.program_id(0) == 0) & (pl.program_id(1) == 0)

    @pl.when(first)
    def _():
        stats_ref[...] = st

    @pl.when(jnp.logical_not(first))
    def _():
        stats_ref[...] += st


def _k3_body(y1_ref, a1_ref, c1_ref, W2_ref, y2_ref, stats_ref):
    z = jnp.maximum(y1_ref[0] * a1_ref[...] + c1_ref[...], 0.0)
    y2 = _dotT(z, W2_ref[...])
    y2_ref[0] = y2
    st = jnp.concatenate([jnp.sum(y2, axis=0)[None, :],
                          jnp.sum(y2 * y2, axis=0)[None, :]], axis=0)
    first = (pl.program_id(0) == 0) & (pl.program_id(1) == 0)

    @pl.when(first)
    def _():
        stats_ref[...] = st

    @pl.when(jnp.logical_not(first))
    def _():
        stats_ref[...] += st


def _k4_body(y2_ref, a2_ref, c2_ref, out_ref):
    out_ref[0] = jnp.maximum(y2_ref[0] * a2_ref[...] + c2_ref[...], 0.0)


def kernel(target_xyz, source_xyz, source_features, target_skip_features,
           W1, g1, b1, W2, g2, b2):
    B, N, _ = target_xyz.shape
    S = source_xyz.shape[1]
    C2 = source_features.shape[2]
    C1 = target_skip_features.shape[2]
    nblk = N // NB
    PTS = B * N
    sxT = jnp.transpose(source_xyz, (0, 2, 1))

    gidx, wn = pl.pallas_call(
        _k1_body,
        grid=(B, nblk),
        in_specs=[
            pl.BlockSpec((1, NB, 3), lambda b, n: (b, n, 0)),
            pl.BlockSpec((1, 3, S), lambda b, n: (b, 0, 0)),
        ],
        out_specs=[
            pl.BlockSpec((1, NB, 3), lambda b, n: (b, n, 0)),
            pl.BlockSpec((1, NB, 3), lambda b, n: (b, n, 0)),
        ],
        out_shape=[
            jax.ShapeDtypeStruct((B, N, 3), jnp.int32),
            jax.ShapeDtypeStruct((B, N, 3), jnp.float32),
        ],
    )(target_xyz, sxT)

    sc_gather = _make_sc_gather(PTS, C2, 32)
    rows = sc_gather(gidx.reshape(-1, GP * 3),
                     source_features.reshape(B * S, C2))
    rows = rows.reshape(B, N, 3 * C2)

    y1, st1 = pl.pallas_call(
        _k2_body,
        grid=(B, nblk),
        in_specs=[
            pl.BlockSpec((1, NB, 3 * C2), lambda b, n: (b, n, 0)),
            pl.BlockSpec((1, NB, 3), lambda b, n: (b, n, 0)),
            pl.BlockSpec((1, NB, C1), lambda b, n: (b, n, 0)),
            pl.BlockSpec((256, 384), lambda b, n: (0, 0)),
        ],
        out_specs=[
            pl.BlockSpec((1, NB, 256), lambda b, n: (b, n, 0)),
            pl.BlockSpec((2, 256), lambda b, n: (0, 0)),
        ],
        out_shape=[
            jax.ShapeDtypeStruct((B, N, 256), jnp.float32),
            jax.ShapeDtypeStruct((2, 256), jnp.float32),
        ],
    )(rows, wn, target_skip_features, W1)

    cnt = float(B * N)
    mean1 = st1[0] / cnt
    var1 = st1[1] / cnt - mean1 * mean1
    a1 = g1 * jax.lax.rsqrt(var1 + 1e-5)
    c1 = b1 - mean1 * a1

    y2, st2 = pl.pallas_call(
        _k3_body,
        grid=(B, nblk),
        in_specs=[
            pl.BlockSpec((1, NB, 256), lambda b, n: (b, n, 0)),
            pl.BlockSpec((1, 256), lambda b, n: (0, 0)),
            pl.BlockSpec((1, 256), lambda b, n: (0, 0)),
            pl.BlockSpec((128, 256), lambda b, n: (0, 0)),
        ],
        out_specs=[
            pl.BlockSpec((1, NB, 128), lambda b, n: (b, n, 0)),
            pl.BlockSpec((2, 128), lambda b, n: (0, 0)),
        ],
        out_shape=[
            jax.ShapeDtypeStruct((B, N, 128), jnp.float32),
            jax.ShapeDtypeStruct((2, 128), jnp.float32),
        ],
    )(y1, a1[None, :], c1[None, :], W2)

    mean2 = st2[0] / cnt
    var2 = st2[1] / cnt - mean2 * mean2
    a2 = g2 * jax.lax.rsqrt(var2 + 1e-5)
    c2 = b2 - mean2 * a2

    out = pl.pallas_call(
        _k4_body,
        grid=(B, nblk),
        in_specs=[
            pl.BlockSpec((1, NB, 128), lambda b, n: (b, n, 0)),
            pl.BlockSpec((1, 128), lambda b, n: (0, 0)),
            pl.BlockSpec((1, 128), lambda b, n: (0, 0)),
        ],
        out_specs=pl.BlockSpec((1, NB, 128), lambda b, n: (b, n, 0)),
        out_shape=jax.ShapeDtypeStruct((B, N, 128), jnp.float32),
    )(y2, a2[None, :], c2[None, :])
    return out


# speedup vs baseline: 1.8608x; 1.0472x over previous
import functools

import jax
import jax.numpy as jnp
from jax import lax
from jax.experimental import pallas as pl
from jax.experimental.pallas import tpu as pltpu
from jax.experimental.pallas import tpu_sc as plsc

NB = 2048
GP = 32


def _dotT(x, w):
    return jax.lax.dot_general(x, w, (((1,), (1,)), ((), ())),
                               preferred_element_type=jnp.float32)


def _k1_body(tx_ref, sxT_ref, idx_ref, w_ref):
    S = sxT_ref.shape[2]
    NBb = tx_ref.shape[1]
    t = tx_ref[0]
    s = sxT_ref[0]
    cross = jnp.dot(t, s, preferred_element_type=jnp.float32)
    t2 = jnp.sum(t * t, axis=1, keepdims=True)
    s2 = jnp.sum(s * s, axis=0, keepdims=True)
    ds = jnp.clip(t2 + s2 - 2.0 * cross, 0.0, None)
    dd = None
    for c in range(3):
        diff = t[:, c:c + 1] - s[c:c + 1, :]
        dd = diff * diff if dd is None else dd + diff * diff
    iota = jax.lax.broadcasted_iota(jnp.int32, (NBb, S), 1)
    dm = ds
    idxs, ws = [], []
    for k in range(3):
        m = jnp.min(dm, axis=1, keepdims=True)
        ik = jnp.min(jnp.where(dm <= m, iota, S), axis=1, keepdims=True)
        selk = iota == ik
        wd = jnp.sum(jnp.where(selk, dd, 0.0), axis=1, keepdims=True)
        ws.append(1.0 / (wd + 1e-8))
        idxs.append(ik)
        if k < 2:
            dm = jnp.where(selk, jnp.inf, dm)
    rs = ws[0] + ws[1] + ws[2]
    b = pl.program_id(0)
    idx_ref[0] = jnp.concatenate(idxs, axis=1) + b * S
    w_ref[0] = jnp.concatenate([w / rs for w in ws], axis=1)


def _make_sc_gather(PTS, C2, NW):
    PPW = PTS // NW
    NG = PPW // GP
    GPC = GP * 3
    mesh = plsc.VectorSubcoreMesh(core_axis_name="c", subcore_axis_name="s")
    NC = 2

    @functools.partial(
        pl.kernel, mesh=mesh,
        compiler_params=pltpu.CompilerParams(needs_layout_passes=False),
        out_type=jax.ShapeDtypeStruct((PTS * 3, C2), jnp.float32),
        scratch_types=[
            pltpu.VMEM((NG, GPC), jnp.int32),
            pltpu.VMEM((GPC, C2), jnp.float32),
            pltpu.VMEM((GPC, C2), jnp.float32),
            pltpu.SemaphoreType.DMA,
            pltpu.SemaphoreType.DMA,
        ],
    )
    def sc_gather(idx_hbm, f_hbm, out_hbm, idx_v, buf0, buf1, sem0, sem1):
        wid = lax.axis_index("s") * NC + lax.axis_index("c")
        pltpu.sync_copy(idx_hbm.at[pl.ds(wid * NG, NG)], idx_v)
        pltpu.async_copy(f_hbm.at[idx_v.at[0]], buf0, sem0)
        pltpu.async_copy(f_hbm.at[idx_v.at[1]], buf1, sem1)
        obase = wid * NG

        def pair(t, carry):
            g = t * 2
            for b in range(2):
                buf = buf0 if b == 0 else buf1
                sem = sem0 if b == 0 else sem1
                gg = g + b
                pltpu.make_async_copy(f_hbm.at[idx_v.at[0]], buf, sem).wait()
                pltpu.sync_copy(
                    buf, out_hbm.at[pl.ds((obase + gg) * GPC, GPC)])

                @pl.when(gg + 2 < NG)
                def _():
                    pltpu.async_copy(f_hbm.at[idx_v.at[gg + 2]], buf, sem)
            return carry

        lax.fori_loop(0, NG // 2, pair, 0)

    return sc_gather


def _k2_body(rows_ref, w_ref, skip_ref, W1_ref, y1_ref, stats_ref):
    g = rows_ref[0]
    w = w_ref[0]
    interp = (w[:, 0:1] * g[:, :256] + w[:, 1:2] * g[:, 256:512]
              + w[:, 2:3] * g[:, 512:])
    y1 = (_dotT(interp, W1_ref[:, :256])
          + _dotT(skip_ref[0], W1_ref[:, 256:]))
    y1_ref[0] = y1
    st = jnp.concatenate([jnp.sum(y1, axis=0)[None, :],
                          jnp.sum(y1 * y1, axis=0)[None, :]], axis=0)
    first = (pl.program_id(0) == 0) & (pl.program_id(1) == 0)

    @pl.when(first)
    def _():
        stats_ref[...] = st

    @pl.when(jnp.logical_not(first))
    def _():
        stats_ref[...] += st


def _k3_body(y1_ref, a1_ref, c1_ref, W2_ref, y2_ref, stats_ref):
    z = jnp.maximum(y1_ref[0] * a1_ref[...] + c1_ref[...], 0.0)
    y2 = _dotT(z, W2_ref[...])
    y2_ref[0] = y2
    st = jnp.concatenate([jnp.sum(y2, axis=0)[None, :],
                          jnp.sum(y2 * y2, axis=0)[None, :]], axis=0)
    first = (pl.program_id(0) == 0) & (pl.program_id(1) == 0)

    @pl.when(first)
    def _():
        stats_ref[...] = st

    @pl.when(jnp.logical_not(first))
    def _():
        stats_ref[...] += st


def _k4_body(y2_ref, a2_ref, c2_ref, out_ref):
    out_ref[0] = jnp.maximum(y2_ref[0] * a2_ref[...] + c2_ref[...], 0.0)


def kernel(target_xyz, source_xyz, source_features, target_skip_features,
           W1, g1, b1, W2, g2, b2):
    B, N, _ = target_xyz.shape
    S = source_xyz.shape[1]
    C2 = source_features.shape[2]
    C1 = target_skip_features.shape[2]
    nblk = N // NB
    PTS = B * N
    sxT = jnp.transpose(source_xyz, (0, 2, 1))

    gidx, wn = pl.pallas_call(
        _k1_body,
        grid=(B, nblk),
        in_specs=[
            pl.BlockSpec((1, NB, 3), lambda b, n: (b, n, 0)),
            pl.BlockSpec((1, 3, S), lambda b, n: (b, 0, 0)),
        ],
        out_specs=[
            pl.BlockSpec((1, NB, 3), lambda b, n: (b, n, 0)),
            pl.BlockSpec((1, NB, 3), lambda b, n: (b, n, 0)),
        ],
        out_shape=[
            jax.ShapeDtypeStruct((B, N, 3), jnp.int32),
            jax.ShapeDtypeStruct((B, N, 3), jnp.float32),
        ],
    )(target_xyz, sxT)

    sc_gather = _make_sc_gather(PTS, C2, 32)
    rows = sc_gather(gidx.reshape(-1, GP * 3),
                     source_features.reshape(B * S, C2))
    rows = rows.reshape(B, N, 3 * C2)

    y1, st1 = pl.pallas_call(
        _k2_body,
        grid=(B, nblk),
        in_specs=[
            pl.BlockSpec((1, NB, 3 * C2), lambda b, n: (b, n, 0)),
            pl.BlockSpec((1, NB, 3), lambda b, n: (b, n, 0)),
            pl.BlockSpec((1, NB, C1), lambda b, n: (b, n, 0)),
            pl.BlockSpec((256, 384), lambda b, n: (0, 0)),
        ],
        out_specs=[
            pl.BlockSpec((1, NB, 256), lambda b, n: (b, n, 0)),
            pl.BlockSpec((2, 256), lambda b, n: (0, 0)),
        ],
        out_shape=[
            jax.ShapeDtypeStruct((B, N, 256), jnp.float32),
            jax.ShapeDtypeStruct((2, 256), jnp.float32),
        ],
    )(rows, wn, target_skip_features, W1)

    cnt = float(B * N)
    mean1 = st1[0] / cnt
    var1 = st1[1] / cnt - mean1 * mean1
    a1 = g1 * jax.lax.rsqrt(var1 + 1e-5)
    c1 = b1 - mean1 * a1

    y2, st2 = pl.pallas_call(
        _k3_body,
        grid=(B, nblk),
        in_specs=[
            pl.BlockSpec((1, NB, 256), lambda b, n: (b, n, 0)),
            pl.BlockSpec((1, 256), lambda b, n: (0, 0)),
            pl.BlockSpec((1, 256), lambda b, n: (0, 0)),
            pl.BlockSpec((128, 256), lambda b, n: (0, 0)),
        ],
        out_specs=[
            pl.BlockSpec((1, NB, 128), lambda b, n: (b, n, 0)),
            pl.BlockSpec((2, 128), lambda b, n: (0, 0)),
        ],
        out_shape=[
            jax.ShapeDtypeStruct((B, N, 128), jnp.float32),
            jax.ShapeDtypeStruct((2, 128), jnp.float32),
        ],
    )(y1, a1[None, :], c1[None, :], W2)

    mean2 = st2[0] / cnt
    var2 = st2[1] / cnt - mean2 * mean2
    a2 = g2 * jax.lax.rsqrt(var2 + 1e-5)
    c2 = b2 - mean2 * a2

    out = pl.pallas_call(
        _k4_body,
        grid=(B, nblk),
        in_specs=[
            pl.BlockSpec((1, NB, 128), lambda b, n: (b, n, 0)),
            pl.BlockSpec((1, 128), lambda b, n: (0, 0)),
            pl.BlockSpec((1, 128), lambda b, n: (0, 0)),
        ],
        out_specs=pl.BlockSpec((1, NB, 128), lambda b, n: (b, n, 0)),
        out_shape=jax.ShapeDtypeStruct((B, N, 128), jnp.float32),
    )(y2, a2[None, :], c2[None, :])
    return out
